# baseline (speedup 1.0000x reference)
import functools

import jax
import jax.numpy as jnp
from jax.experimental import pallas as pl

N = 10000
E = 320000
D = 128
HID = 256
G = 128
T = 10
MC = 20
TAU = 0.5
GAMMA = 0.4


def _mlp2_body(x_ref, w1_ref, b1_ref, s1_ref, bb1_ref, w2_ref, b2_ref, o_ref):
    z = jnp.dot(x_ref[...], w1_ref[...], preferred_element_type=jnp.float32)
    z = z + b1_ref[...]
    z = s1_ref[...] * z + bb1_ref[...]
    z = jnp.maximum(z, 0.0)
    o = jnp.dot(z, w2_ref[...], preferred_element_type=jnp.float32)
    o_ref[...] = o + b2_ref[...]


def _mlp2(x, w1, b1, s1, bb1, w2, b2):
    n, din = x.shape
    hid = w1.shape[1]
    dout = w2.shape[1]
    bn = n if n <= 1024 else 1000
    assert n % bn == 0
    b1 = b1.reshape(1, hid)
    s1 = s1.reshape(1, hid)
    bb1 = bb1.reshape(1, hid)
    b2 = b2.reshape(1, dout)
    grid = (n // bn,)
    return pl.pallas_call(
        _mlp2_body,
        grid=grid,
        in_specs=[
            pl.BlockSpec((bn, din), lambda i: (i, 0)),
            pl.BlockSpec((din, hid), lambda i: (0, 0)),
            pl.BlockSpec((1, hid), lambda i: (0, 0)),
            pl.BlockSpec((1, hid), lambda i: (0, 0)),
            pl.BlockSpec((1, hid), lambda i: (0, 0)),
            pl.BlockSpec((hid, dout), lambda i: (0, 0)),
            pl.BlockSpec((1, dout), lambda i: (0, 0)),
        ],
        out_specs=pl.BlockSpec((bn, dout), lambda i: (i, 0)),
        out_shape=jax.ShapeDtypeStruct((n, dout), jnp.float32),
    )(x, w1, b1, s1, bb1, w2, b2)


def _gin_encoder(h, src, dst, eps, W1, b1, s1, bb1, W2, b2, os_, ob):
    nl = W1.shape[0]
    for l in range(nl):
        agg = jax.ops.segment_sum(h[src], dst, num_segments=N)
        z = _mlp2((1.0 + eps[l]) * h + agg, W1[l], b1[l], s1[l], bb1[l], W2[l], b2[l])
        z = os_[l] * z + ob[l]
        if l < nl - 1:
            z = jax.nn.relu(z)
        h = h + z
    return h


def _normalize(z):
    return z / jnp.clip(jnp.linalg.norm(z, axis=1, keepdims=True), 1e-12)


def _info_nce(z1, z2, temp):
    z1 = _normalize(z1)
    z2 = _normalize(z2)
    logits = (z1 @ z2.T) / temp
    labels = jnp.arange(z1.shape[0])
    lse = jax.nn.logsumexp(logits, axis=1)
    return jnp.mean(lse - logits[labels, labels])


def kernel(x, edge_index, batch,
           enc_eps, enc_W1, enc_b1, enc_s1, enc_bb1, enc_W2, enc_b2, enc_os, enc_ob,
           rat_eps, rat_W1, rat_b1, rat_s1, rat_bb1, rat_W2, rat_b2, rat_os, rat_ob,
           gate_W1, gate_b1, gate_s1, gate_bb1, gate_W2, gate_b2,
           pred_W1, pred_b1, pred_s1, pred_bb1, pred_W2, pred_b2,
           dec_W1, dec_b1, dec_s1, dec_bb1, dec_W2, dec_b2,
           sel_W, sel_b):
    src, dst = edge_index[0], edge_index[1]

    h_node = _gin_encoder(x, src, dst, enc_eps, enc_W1, enc_b1, enc_s1, enc_bb1,
                          enc_W2, enc_b2, enc_os, enc_ob)

    logits = x @ sel_W + sel_b
    samples = []
    for i in range(MC):
        k = jax.random.fold_in(jax.random.key(42), i)
        u = jax.random.uniform(k, logits.shape, jnp.float32, 1e-6, 1.0 - 1e-6)
        gmb = -jnp.log(-jnp.log(u))
        y = jax.nn.softmax(logits + gmb, axis=1)
        samples.append(y[:, 1:2])
    samp = jnp.stack(samples, 0)
    mean_score = samp.mean(0)
    var = jnp.var(samp, axis=0, ddof=1)
    mask = jnp.where(var < 0.05, mean_score, mean_score * (0.05 / (var + 1e-8)))

    h_masked = _gin_encoder(x * mask, src, dst, enc_eps, enc_W1, enc_b1, enc_s1,
                            enc_bb1, enc_W2, enc_b2, enc_os, enc_ob)
    motif_pred = _mlp2(h_masked, dec_W1, dec_b1, dec_s1, dec_bb1, dec_W2, dec_b2)
    denom = jnp.clip(jnp.linalg.norm(motif_pred, axis=1) * jnp.linalg.norm(h_node, axis=1), 1e-8)
    cs = jnp.sum(motif_pred * h_node, axis=1) / denom
    loss_recon = 1.0 - jnp.mean(cs)

    x_rat = _gin_encoder(x, src, dst, rat_eps, rat_W1, rat_b1, rat_s1, rat_bb1,
                         rat_W2, rat_b2, rat_os, rat_ob)
    gate = jax.nn.sigmoid(_mlp2(x_rat, gate_W1, gate_b1, gate_s1, gate_bb1, gate_W2, gate_b2))
    h_r = jax.ops.segment_sum(gate * h_node, batch, num_segments=G)
    r_num = jax.ops.segment_sum(gate, batch, num_segments=G) + 1e-8
    env_num = jax.ops.segment_sum(1.0 - gate, batch, num_segments=G) + 1e-8
    pred_rem = _mlp2(h_r, pred_W1, pred_b1, pred_s1, pred_bb1, pred_W2, pred_b2)
    contrast = (_info_nce(h_r, h_r, TAU) + _info_nce(h_node, h_masked, TAU)) / 2.0
    loss_reg = jnp.mean(jnp.abs(r_num / (r_num + env_num) - GAMMA))
    return pred_rem, contrast, loss_reg, loss_recon


# speedup vs baseline: 1.9669x; 1.9669x over previous
import functools

import jax
import jax.numpy as jnp
from jax import lax
from jax.experimental import pallas as pl
from jax.experimental.pallas import tpu as pltpu
from jax.experimental.pallas import tpu_sc as plsc

N = 10000
E = 320000
D = 128
HID = 256
G = 128
T = 10
MC = 20
TAU = 0.5
GAMMA = 0.4

NC = 2
NS = 16
K = 128
EPT = E // NS
CH = -(-EPT // K) + ((-(-EPT // K)) % 2)
EPADT = CH * K
HALF = N // 2
NP = 5120
SLAB = NP // NS
TRASH = HALF
BN = 1000


def _segsum_body(h_hbm, src_hbm, dst_hbm, zeros_hbm, out_hbm,
                 sidx, didx, rows, acc, gsem0, gsem1):
    c = lax.axis_index("c")
    s = lax.axis_index("s")

    pltpu.sync_copy(zeros_hbm, acc.at[pl.ds(s * SLAB, SLAB)])
    pltpu.sync_copy(src_hbm.at[s], sidx)
    pltpu.sync_copy(dst_hbm.at[c].at[s], didx)
    plsc.subcore_barrier()

    pltpu.async_copy(h_hbm.at[sidx.at[0]], rows.at[0], gsem0)
    pltpu.async_copy(h_hbm.at[sidx.at[1]], rows.at[1], gsem1)

    def step(i, _):
        j0 = 2 * i
        pltpu.make_async_copy(h_hbm.at[sidx.at[j0]], rows.at[0], gsem0).wait()
        pltpu.sync_copy(rows.at[0], acc.at[didx.at[j0]], add=True)
        pltpu.async_copy(h_hbm.at[sidx.at[j0 + 2]], rows.at[0], gsem0)
        pltpu.make_async_copy(h_hbm.at[sidx.at[j0]], rows.at[1], gsem1).wait()
        pltpu.sync_copy(rows.at[1], acc.at[didx.at[j0 + 1]], add=True)
        pltpu.async_copy(h_hbm.at[sidx.at[j0 + 3]], rows.at[1], gsem1)
        return 0

    lax.fori_loop(0, CH // 2 - 1, step, 0)
    j0 = CH - 2
    pltpu.make_async_copy(h_hbm.at[sidx.at[j0]], rows.at[0], gsem0).wait()
    pltpu.sync_copy(rows.at[0], acc.at[didx.at[j0]], add=True)
    pltpu.make_async_copy(h_hbm.at[sidx.at[j0]], rows.at[1], gsem1).wait()
    pltpu.sync_copy(rows.at[1], acc.at[didx.at[j0 + 1]], add=True)

    plsc.subcore_barrier()
    pltpu.sync_copy(acc.at[pl.ds(s * SLAB, SLAB)],
                    out_hbm.at[c, pl.ds(s * SLAB, SLAB)])


def _make_segsum():
    mesh = plsc.VectorSubcoreMesh(core_axis_name="c", subcore_axis_name="s",
                                  num_cores=NC, num_subcores=NS)
    return pl.kernel(
        _segsum_body,
        out_type=jax.ShapeDtypeStruct((NC, NP, D), jnp.float32),
        mesh=mesh,
        scratch_types=[
            pltpu.VMEM((CH, K), jnp.int32),
            pltpu.VMEM((CH, K), jnp.int32),
            pltpu.VMEM((2, K, D), jnp.float32),
            pltpu.VMEM_SHARED((NP, D), jnp.float32),
            pltpu.SemaphoreType.DMA,
            pltpu.SemaphoreType.DMA,
        ],
    )


def _layer_body(last, h_ref, p_ref, eps_ref, w1_ref, b1_ref, w2_ref, b2_ref,
                o_ref):
    h = h_ref[...]
    a = (1.0 + eps_ref[0, 0]) * h + p_ref[0]
    t = jnp.dot(a, w1_ref[...], preferred_element_type=jnp.float32) + b1_ref[...]
    t = jnp.maximum(t, 0.0)
    z = jnp.dot(t, w2_ref[...], preferred_element_type=jnp.float32) + b2_ref[...]
    if not last:
        z = jnp.maximum(z, 0.0)
    o_ref[...] = h + z


def _layer(h, parts, eps, w1f, b1f, w2f, b2f, last):
    return pl.pallas_call(
        functools.partial(_layer_body, last),
        grid=(N // BN,),
        in_specs=[
            pl.BlockSpec((BN, D), lambda i: (i, 0)),
            pl.BlockSpec((1, BN, D), lambda i: (i // 5, i % 5, 0)),
            pl.BlockSpec((1, 1), lambda i: (0, 0)),
            pl.BlockSpec((D, HID), lambda i: (0, 0)),
            pl.BlockSpec((1, HID), lambda i: (0, 0)),
            pl.BlockSpec((HID, D), lambda i: (0, 0)),
            pl.BlockSpec((1, D), lambda i: (0, 0)),
        ],
        out_specs=pl.BlockSpec((BN, D), lambda i: (i, 0)),
        out_shape=jax.ShapeDtypeStruct((N, D), jnp.float32),
    )(h, parts, eps, w1f, b1f, w2f, b2f)


def _mask_body(x_ref, dg_ref, dw_ref, db_ref, o_ref):
    x = x_ref[...]
    dlog = jnp.dot(x, dw_ref[...], preferred_element_type=jnp.float32) + db_ref[0, 0]
    s = jax.nn.sigmoid(dlog + dg_ref[...])
    mean = jnp.mean(s, axis=1, keepdims=True)
    var = jnp.sum((s - mean) ** 2, axis=1, keepdims=True) / (MC - 1)
    mask = jnp.where(var < 0.05, mean, mean * (0.05 / (var + 1e-8)))
    o_ref[...] = x * mask


def _masked_x(x, dg, dw, db):
    return pl.pallas_call(
        _mask_body,
        grid=(N // BN,),
        in_specs=[
            pl.BlockSpec((BN, D), lambda i: (i, 0)),
            pl.BlockSpec((BN, MC), lambda i: (i, 0)),
            pl.BlockSpec((D, 1), lambda i: (0, 0)),
            pl.BlockSpec((1, 1), lambda i: (0, 0)),
        ],
        out_specs=pl.BlockSpec((BN, D), lambda i: (i, 0)),
        out_shape=jax.ShapeDtypeStruct((N, D), jnp.float32),
    )(x, dg, dw, db)


def _normalize_body(x_ref, o_ref):
    x = x_ref[...]
    n = jnp.sqrt(jnp.sum(x * x, axis=1, keepdims=True))
    o_ref[...] = x / jnp.maximum(n, 1e-12)


def _normalize_rows(x):
    n = x.shape[0]
    bn = min(n, BN)
    return pl.pallas_call(
        _normalize_body,
        grid=(n // bn,),
        in_specs=[pl.BlockSpec((bn, D), lambda i: (i, 0))],
        out_specs=pl.BlockSpec((bn, D), lambda i: (i, 0)),
        out_shape=jax.ShapeDtypeStruct((n, D), jnp.float32),
    )(x)


def _nce_body(h1_ref, z2f_ref, z2b_ref, o_ref):
    h1 = h1_ref[...]
    n1 = jnp.sqrt(jnp.sum(h1 * h1, axis=1, keepdims=True))
    z1 = h1 / jnp.maximum(n1, 1e-12)
    cb = 2000
    m = jnp.full((BN, 1), -jnp.inf, jnp.float32)
    ssum = jnp.zeros((BN, 1), jnp.float32)
    for c in range(N // cb):
        z2c = z2f_ref[pl.ds(c * cb, cb), :]
        l = lax.dot_general(z1, z2c, (((1,), (1,)), ((), ())),
                            preferred_element_type=jnp.float32) / TAU
        cm = jnp.max(l, axis=1, keepdims=True)
        nm = jnp.maximum(m, cm)
        ssum = ssum * jnp.exp(m - nm) + jnp.sum(jnp.exp(l - nm), axis=1, keepdims=True)
        m = nm
    lse = jnp.log(ssum) + m
    diag = jnp.sum(z1 * z2b_ref[...], axis=1, keepdims=True) / TAU
    o_ref[...] = lse - diag


def _nce_rows(h1, z2n):
    return pl.pallas_call(
        _nce_body,
        grid=(N // BN,),
        in_specs=[
            pl.BlockSpec((BN, D), lambda i: (i, 0)),
            pl.BlockSpec((N, D), lambda i: (0, 0)),
            pl.BlockSpec((BN, D), lambda i: (i, 0)),
        ],
        out_specs=pl.BlockSpec((BN, 1), lambda i: (i, 0)),
        out_shape=jax.ShapeDtypeStruct((N, 1), jnp.float32),
    )(h1, z2n, z2n)


def _dec_cos_body(hm_ref, hn_ref, w1_ref, b1_ref, w2_ref, b2_ref, o_ref):
    t = jnp.dot(hm_ref[...], w1_ref[...], preferred_element_type=jnp.float32) + b1_ref[...]
    t = jnp.maximum(t, 0.0)
    mp = jnp.dot(t, w2_ref[...], preferred_element_type=jnp.float32) + b2_ref[...]
    hn = hn_ref[...]
    num = jnp.sum(mp * hn, axis=1, keepdims=True)
    den = jnp.sqrt(jnp.sum(mp * mp, axis=1, keepdims=True)) * \
        jnp.sqrt(jnp.sum(hn * hn, axis=1, keepdims=True))
    o_ref[...] = num / jnp.maximum(den, 1e-8)


def _dec_cos(hm, hn, w1f, b1f, w2f, b2f):
    return pl.pallas_call(
        _dec_cos_body,
        grid=(N // BN,),
        in_specs=[
            pl.BlockSpec((BN, D), lambda i: (i, 0)),
            pl.BlockSpec((BN, D), lambda i: (i, 0)),
            pl.BlockSpec((D, HID), lambda i: (0, 0)),
            pl.BlockSpec((1, HID), lambda i: (0, 0)),
            pl.BlockSpec((HID, D), lambda i: (0, 0)),
            pl.BlockSpec((1, D), lambda i: (0, 0)),
        ],
        out_specs=pl.BlockSpec((BN, 1), lambda i: (i, 0)),
        out_shape=jax.ShapeDtypeStruct((N, 1), jnp.float32),
    )(hm, hn, w1f, b1f, w2f, b2f)


def _pool_body(xr_ref, hn_ref, b_ref, w1_ref, b1_ref, w2_ref, b2_ref,
               hr_ref, rn_ref, en_ref):
    i = pl.program_id(0)

    @pl.when(i == 0)
    def _():
        hr_ref[...] = jnp.zeros_like(hr_ref)
        rn_ref[...] = jnp.zeros_like(rn_ref)
        en_ref[...] = jnp.zeros_like(en_ref)

    t = jnp.dot(xr_ref[...], w1_ref[...], preferred_element_type=jnp.float32) + b1_ref[...]
    t = jnp.maximum(t, 0.0)
    gate = jax.nn.sigmoid(
        jnp.dot(t, w2_ref[...], preferred_element_type=jnp.float32) + b2_ref[...])
    onehot = (b_ref[...] == lax.broadcasted_iota(jnp.int32, (BN, G), 1)
              ).astype(jnp.float32)
    hr_ref[...] += lax.dot_general(onehot, gate * hn_ref[...],
                                   (((0,), (0,)), ((), ())),
                                   preferred_element_type=jnp.float32)
    rn_ref[...] += lax.dot_general(onehot, gate, (((0,), (0,)), ((), ())),
                                   preferred_element_type=jnp.float32)
    en_ref[...] += lax.dot_general(onehot, 1.0 - gate, (((0,), (0,)), ((), ())),
                                   preferred_element_type=jnp.float32)


def _gate_pool(x_rat, h_node, batch2d, w1f, b1f, w2f, b2f):
    return pl.pallas_call(
        _pool_body,
        grid=(N // BN,),
        in_specs=[
            pl.BlockSpec((BN, D), lambda i: (i, 0)),
            pl.BlockSpec((BN, D), lambda i: (i, 0)),
            pl.BlockSpec((BN, 1), lambda i: (i, 0)),
            pl.BlockSpec((D, HID), lambda i: (0, 0)),
            pl.BlockSpec((1, HID), lambda i: (0, 0)),
            pl.BlockSpec((HID, 1), lambda i: (0, 0)),
            pl.BlockSpec((1, 1), lambda i: (0, 0)),
        ],
        out_specs=[
            pl.BlockSpec((G, D), lambda i: (0, 0)),
            pl.BlockSpec((G, 1), lambda i: (0, 0)),
            pl.BlockSpec((G, 1), lambda i: (0, 0)),
        ],
        out_shape=[
            jax.ShapeDtypeStruct((G, D), jnp.float32),
            jax.ShapeDtypeStruct((G, 1), jnp.float32),
            jax.ShapeDtypeStruct((G, 1), jnp.float32),
        ],
    )(x_rat, h_node, batch2d, w1f, b1f, w2f, b2f)


def _final_body(hr_ref, rn_ref, en_ref, w1_ref, b1_ref, w2_ref, b2_ref,
                pred_ref, nce_ref, reg_ref):
    hr = hr_ref[...]
    t = jnp.dot(hr, w1_ref[...], preferred_element_type=jnp.float32) + b1_ref[...]
    t = jnp.maximum(t, 0.0)
    pred_ref[...] = jnp.dot(t, w2_ref[...], preferred_element_type=jnp.float32) + b2_ref[...]

    n = jnp.sqrt(jnp.sum(hr * hr, axis=1, keepdims=True))
    z = hr / jnp.maximum(n, 1e-12)
    l = lax.dot_general(z, z, (((1,), (1,)), ((), ())),
                        preferred_element_type=jnp.float32) / TAU
    m = jnp.max(l, axis=1, keepdims=True)
    lse = jnp.log(jnp.sum(jnp.exp(l - m), axis=1, keepdims=True)) + m
    diag = jnp.sum(z * z, axis=1, keepdims=True) / TAU
    nce_ref[...] = jnp.mean(lse - diag).reshape(1, 1)

    rn = rn_ref[...] + 1e-8
    en = en_ref[...] + 1e-8
    reg_ref[...] = jnp.mean(jnp.abs(rn / (rn + en) - GAMMA)).reshape(1, 1)


def _finals(h_r, rn, en, w1f, b1f, w2f, b2f):
    return pl.pallas_call(
        _final_body,
        grid=(1,),
        in_specs=[
            pl.BlockSpec((G, D), lambda i: (0, 0)),
            pl.BlockSpec((G, 1), lambda i: (0, 0)),
            pl.BlockSpec((G, 1), lambda i: (0, 0)),
            pl.BlockSpec((D, HID), lambda i: (0, 0)),
            pl.BlockSpec((1, HID), lambda i: (0, 0)),
            pl.BlockSpec((HID, T), lambda i: (0, 0)),
            pl.BlockSpec((1, T), lambda i: (0, 0)),
        ],
        out_specs=[
            pl.BlockSpec((G, T), lambda i: (0, 0)),
            pl.BlockSpec((1, 1), lambda i: (0, 0)),
            pl.BlockSpec((1, 1), lambda i: (0, 0)),
        ],
        out_shape=[
            jax.ShapeDtypeStruct((G, T), jnp.float32),
            jax.ShapeDtypeStruct((1, 1), jnp.float32),
            jax.ShapeDtypeStruct((1, 1), jnp.float32),
        ],
    )(h_r, rn, en, w1f, b1f, w2f, b2f)


def _fold_inner(w1, b1, s1, bb1):
    return w1 * s1[None, :], (b1 * s1 + bb1).reshape(1, -1)


def _fold_outer(w2, b2, os_, ob):
    return w2 * os_[None, :], (b2 * os_ + ob).reshape(1, -1)


def _encoder(segsum, h, eps, W1, b1, s1, bb1, W2, b2, os_, ob):
    nl = W1.shape[0]
    for l in range(nl):
        parts = segsum(h)
        w1f, b1f = _fold_inner(W1[l], b1[l], s1[l], bb1[l])
        w2f, b2f = _fold_outer(W2[l], b2[l], os_[l], ob[l])
        h = _layer(h, parts, eps[l].reshape(1, 1), w1f, b1f, w2f, b2f,
                   last=(l == nl - 1))
    return h


def kernel(x, edge_index, batch,
           enc_eps, enc_W1, enc_b1, enc_s1, enc_bb1, enc_W2, enc_b2, enc_os, enc_ob,
           rat_eps, rat_W1, rat_b1, rat_s1, rat_bb1, rat_W2, rat_b2, rat_os, rat_ob,
           gate_W1, gate_b1, gate_s1, gate_bb1, gate_W2, gate_b2,
           pred_W1, pred_b1, pred_s1, pred_bb1, pred_W2, pred_b2,
           dec_W1, dec_b1, dec_s1, dec_bb1, dec_W2, dec_b2,
           sel_W, sel_b):
    src, dst = edge_index[0], edge_index[1]
    pad = NS * EPADT - E
    src_t = jnp.concatenate([src, jnp.zeros((pad,), jnp.int32)]).reshape(NS, CH, K)
    dst_p = jnp.concatenate([dst, jnp.full((pad,), N, jnp.int32)])
    dst_c0 = jnp.where(dst_p < HALF, dst_p, TRASH)
    dst_c1 = jnp.where((dst_p >= HALF) & (dst_p < N), dst_p - HALF, TRASH)
    dst_t = jnp.stack([dst_c0, dst_c1]).reshape(NC, NS, CH, K)
    zeros = jnp.zeros((SLAB, D), jnp.float32)
    segsum_k = _make_segsum()
    segsum = lambda h: segsum_k(h, src_t, dst_t, zeros)

    gs = []
    for i in range(MC):
        k = jax.random.fold_in(jax.random.key(42), i)
        u = jax.random.uniform(k, (N, 2), jnp.float32, 1e-6, 1.0 - 1e-6)
        g = -jnp.log(-jnp.log(u))
        gs.append(g[:, 1] - g[:, 0])
    dg = jnp.stack(gs, axis=1)
    dw = (sel_W[:, 1] - sel_W[:, 0]).reshape(D, 1)
    db = (sel_b[1] - sel_b[0]).reshape(1, 1)

    h_node = _encoder(segsum, x, enc_eps, enc_W1, enc_b1, enc_s1,
                      enc_bb1, enc_W2, enc_b2, enc_os, enc_ob)
    xm = _masked_x(x, dg, dw, db)
    h_masked = _encoder(segsum, xm, enc_eps, enc_W1, enc_b1, enc_s1,
                        enc_bb1, enc_W2, enc_b2, enc_os, enc_ob)
    x_rat = _encoder(segsum, x, rat_eps, rat_W1, rat_b1, rat_s1,
                     rat_bb1, rat_W2, rat_b2, rat_os, rat_ob)

    dw1f, db1f = _fold_inner(dec_W1, dec_b1, dec_s1, dec_bb1)
    cs = _dec_cos(h_masked, h_node, dw1f, db1f, dec_W2, dec_b2.reshape(1, -1))
    loss_recon = 1.0 - jnp.mean(cs)

    gw1f, gb1f = _fold_inner(gate_W1, gate_b1, gate_s1, gate_bb1)
    h_r, rn, en = _gate_pool(x_rat, h_node, batch.reshape(N, 1), gw1f, gb1f,
                             gate_W2, gate_b2.reshape(1, 1))

    pw1f, pb1f = _fold_inner(pred_W1, pred_b1, pred_s1, pred_bb1)
    pred_rem, nce_small, loss_reg2d = _finals(h_r, rn, en, pw1f, pb1f,
                                              pred_W2, pred_b2.reshape(1, -1))
    loss_reg = loss_reg2d[0, 0]

    z2n = _normalize_rows(h_masked)
    v = _nce_rows(h_node, z2n)
    contrast = (nce_small[0, 0] + jnp.mean(v)) / 2.0

    return pred_rem, contrast, loss_reg, loss_recon
